# baseline (speedup 1.0000x reference)
import jax
import jax.numpy as jnp
from jax.experimental import pallas as pl
from jax.experimental.pallas import tpu as pltpu

N = 10000
S = 16
D = 256
HALF = 128
BLOCK_N = 400


def _sage_body(self_ref, neigh_ref, ws_ref, wn_ref, out_ref):
    acc = neigh_ref[...]
    width = S * D
    while width > D:
        half = width // 2
        acc = acc[:, :half] + acc[:, half:width]
        width = half
    neigh_mean = acc * (1.0 / S)
    from_self = jnp.dot(self_ref[...], ws_ref[...],
                        preferred_element_type=jnp.float32)
    from_neigh = jnp.dot(neigh_mean, wn_ref[...],
                         preferred_element_type=jnp.float32)
    out_ref[...] = jnp.maximum(
        jnp.concatenate([from_self, from_neigh], axis=-1), 0.0)


def kernel(self_vectors, neigh_vectors, W_self, W_neigh):
    neigh_flat = neigh_vectors.reshape(N, S * D)
    grid = (N // BLOCK_N,)
    return pl.pallas_call(
        _sage_body,
        grid=grid,
        in_specs=[
            pl.BlockSpec((BLOCK_N, D), lambda i: (i, 0)),
            pl.BlockSpec((BLOCK_N, S * D), lambda i: (i, 0)),
            pl.BlockSpec((D, HALF), lambda i: (0, 0)),
            pl.BlockSpec((D, HALF), lambda i: (0, 0)),
        ],
        out_specs=pl.BlockSpec((BLOCK_N, 2 * HALF), lambda i: (i, 0)),
        out_shape=jax.ShapeDtypeStruct((N, 2 * HALF), jnp.float32),
        compiler_params=pltpu.CompilerParams(
            dimension_semantics=("arbitrary",),
        ),
    )(self_vectors, neigh_flat, W_self, W_neigh)


# speedup vs baseline: 3.1835x; 3.1835x over previous
import jax
import jax.numpy as jnp
from jax.experimental import pallas as pl
from jax.experimental.pallas import tpu as pltpu

N = 10000
S = 16
D = 256
HALF = 128
BLOCK_N = 1000


def _sage_body(self_ref, neigh_ref, ws_ref, wn_ref, out_ref):
    neigh_mean = jnp.sum(neigh_ref[...], axis=1) * (1.0 / S)
    from_self = jnp.dot(self_ref[...], ws_ref[...],
                        preferred_element_type=jnp.float32)
    from_neigh = jnp.dot(neigh_mean, wn_ref[...],
                         preferred_element_type=jnp.float32)
    out_ref[...] = jnp.maximum(
        jnp.concatenate([from_self, from_neigh], axis=-1), 0.0)


def kernel(self_vectors, neigh_vectors, W_self, W_neigh):
    grid = (N // BLOCK_N,)
    return pl.pallas_call(
        _sage_body,
        grid=grid,
        in_specs=[
            pl.BlockSpec((BLOCK_N, D), lambda i: (i, 0)),
            pl.BlockSpec((BLOCK_N, S, D), lambda i: (i, 0, 0)),
            pl.BlockSpec((D, HALF), lambda i: (0, 0)),
            pl.BlockSpec((D, HALF), lambda i: (0, 0)),
        ],
        out_specs=pl.BlockSpec((BLOCK_N, 2 * HALF), lambda i: (i, 0)),
        out_shape=jax.ShapeDtypeStruct((N, 2 * HALF), jnp.float32),
        compiler_params=pltpu.CompilerParams(
            dimension_semantics=("arbitrary",),
        ),
    )(self_vectors, neigh_vectors, W_self, W_neigh)
